# baseline (speedup 1.0000x reference)
import numpy as np
import jax
import jax.numpy as jnp
from jax import lax
from jax.experimental import pallas as pl
from jax.experimental.pallas import tpu as pltpu

SR = 16000
WIN = 400
HOP = 160
NFFT = 512
NMELS = 80
NQUANT = 256
L_ENC = 320
L_DEC = 2047

B = 16
T = 16384
NFRAMES = 1 + (T - WIN) // HOP
TDEC = T - 2 * L_ENC
NBINS = NFFT // 2 + 1
TB = 15744


def _mel_fb_np():
    def h2m(f):
        return 2595.0 * np.log10(1.0 + f / 700.0)

    def m2h(m):
        return 700.0 * (10.0 ** (m / 2595.0) - 1.0)

    pts = np.linspace(h2m(0.0), h2m(SR / 2.0), NMELS + 2)
    hz = m2h(pts)
    bins = np.floor((NFFT + 1) * hz / SR).astype(int)
    fb = np.zeros((NMELS, NBINS), dtype=np.float32)
    for i in range(1, NMELS + 1):
        l, c, r = bins[i - 1], bins[i], bins[i + 1]
        for j in range(l, c):
            fb[i - 1, j] = (j - l) / max(c - l, 1)
        for j in range(c, min(r, NBINS)):
            fb[i - 1, j] = (r - j) / max(r - c, 1)
    return fb


def _dft_mats_np():
    w = np.hanning(WIN).astype(np.float64)
    n = np.arange(WIN, dtype=np.float64)
    k = np.arange(NBINS, dtype=np.float64)
    ang = 2.0 * np.pi * np.outer(n, k) / NFFT
    cr = np.cos(ang) * w[:, None]
    ci = np.sin(ang) * w[:, None]
    crp = np.zeros((3 * HOP, NBINS))
    cip = np.zeros((3 * HOP, NBINS))
    crp[:WIN] = cr
    cip[:WIN] = ci
    return (crp.reshape(3, HOP, NBINS).astype(np.float32),
            cip.reshape(3, HOP, NBINS).astype(np.float32))


_FB_NP = _mel_fb_np()
_WR_NP, _WI_NP = _dft_mats_np()


def _mels_body(wav3_ref, wr_ref, wi_ref, fb_ref, out_ref):
    a = wav3_ref[0]
    a0 = a[0:NFRAMES]
    a1 = a[1:NFRAMES + 1]
    a2 = a[2:NFRAMES + 2]
    f32 = jnp.float32
    re = (jnp.dot(a0, wr_ref[0], preferred_element_type=f32)
          + jnp.dot(a1, wr_ref[1], preferred_element_type=f32)
          + jnp.dot(a2, wr_ref[2], preferred_element_type=f32))
    im = (jnp.dot(a0, wi_ref[0], preferred_element_type=f32)
          + jnp.dot(a1, wi_ref[1], preferred_element_type=f32)
          + jnp.dot(a2, wi_ref[2], preferred_element_type=f32))
    spec = re * re + im * im
    melt = lax.dot_general(fb_ref[...], spec,
                           (((1,), (1,)), ((), ())),
                           preferred_element_type=f32)
    out_ref[0] = jnp.log(melt + 1e-6)


def _onehot_body(wavd_ref, oh_ref, code_ref):
    mu = NQUANT - 1
    x = wavd_ref[0]
    xc = jnp.clip(x, -1.0, 1.0)
    amp = jnp.sign(xc) * jnp.log1p(mu * jnp.abs(xc)) / np.log1p(mu)
    code = jnp.floor((amp + 1.0) * 0.5 * mu + 0.5).astype(jnp.int32)
    code_ref[0] = code
    q = lax.broadcasted_iota(jnp.int32, (NQUANT, TB), 0)
    oh_ref[0] = jnp.where(q == code, 1.0, 0.0).astype(jnp.float32)


def kernel(inds_np, wav_np, quant_onehot):
    wav3 = wav_np[:, :102 * HOP].reshape(B, 102, HOP)
    mels = pl.pallas_call(
        _mels_body,
        grid=(B,),
        in_specs=[
            pl.BlockSpec((1, 102, HOP), lambda b: (b, 0, 0)),
            pl.BlockSpec((3, HOP, NBINS), lambda b: (0, 0, 0)),
            pl.BlockSpec((3, HOP, NBINS), lambda b: (0, 0, 0)),
            pl.BlockSpec((NMELS, NBINS), lambda b: (0, 0)),
        ],
        out_specs=pl.BlockSpec((1, NMELS, NFRAMES), lambda b: (b, 0, 0)),
        out_shape=jax.ShapeDtypeStruct((B, NMELS, NFRAMES), jnp.float32),
    )(wav3, jnp.asarray(_WR_NP), jnp.asarray(_WI_NP), jnp.asarray(_FB_NP))

    wav_dec = lax.slice(wav_np, (0, L_ENC), (B, T - L_ENC)).reshape(B, 1, TDEC)
    onehot, codes = pl.pallas_call(
        _onehot_body,
        grid=(B, TDEC // TB),
        in_specs=[pl.BlockSpec((1, 1, TB), lambda b, t: (b, 0, t))],
        out_specs=[
            pl.BlockSpec((1, NQUANT, TB), lambda b, t: (b, 0, t)),
            pl.BlockSpec((1, 1, TB), lambda b, t: (b, 0, t)),
        ],
        out_shape=[
            jax.ShapeDtypeStruct((B, NQUANT, TDEC), jnp.float32),
            jax.ShapeDtypeStruct((B, 1, TDEC), jnp.int32),
        ],
    )(wav_dec)

    wav_compand_out = lax.slice(codes, (0, 0, L_DEC), (B, 1, TDEC)).reshape(B, TDEC - L_DEC)
    return (inds_np, mels, onehot, wav_compand_out)


# speedup vs baseline: 1.0822x; 1.0822x over previous
import numpy as np
import jax
import jax.numpy as jnp
from jax import lax
from jax.experimental import pallas as pl
from jax.experimental.pallas import tpu as pltpu

SR = 16000
WIN = 400
HOP = 160
NFFT = 512
NMELS = 80
NQUANT = 256
L_ENC = 320
L_DEC = 2047

B = 16
T = 16384
NFRAMES = 1 + (T - WIN) // HOP
TDEC = T - 2 * L_ENC
NBINS = NFFT // 2 + 1
TB = 5248
NTB = TDEC // TB


def _mel_fb_np():
    def h2m(f):
        return 2595.0 * np.log10(1.0 + f / 700.0)

    def m2h(m):
        return 700.0 * (10.0 ** (m / 2595.0) - 1.0)

    pts = np.linspace(h2m(0.0), h2m(SR / 2.0), NMELS + 2)
    hz = m2h(pts)
    bins = np.floor((NFFT + 1) * hz / SR).astype(int)
    fb = np.zeros((NMELS, NBINS), dtype=np.float32)
    for i in range(1, NMELS + 1):
        l, c, r = bins[i - 1], bins[i], bins[i + 1]
        for j in range(l, c):
            fb[i - 1, j] = (j - l) / max(c - l, 1)
        for j in range(c, min(r, NBINS)):
            fb[i - 1, j] = (r - j) / max(r - c, 1)
    return fb


def _dft_mats_np():
    w = np.hanning(WIN).astype(np.float64)
    n = np.arange(WIN, dtype=np.float64)
    k = np.arange(NBINS, dtype=np.float64)
    ang = 2.0 * np.pi * np.outer(n, k) / NFFT
    cr = np.cos(ang) * w[:, None]
    ci = np.sin(ang) * w[:, None]
    crp = np.zeros((3 * HOP, NBINS))
    cip = np.zeros((3 * HOP, NBINS))
    crp[:WIN] = cr
    cip[:WIN] = ci
    return (crp.reshape(3, HOP, NBINS).astype(np.float32),
            cip.reshape(3, HOP, NBINS).astype(np.float32))


_FB_NP = _mel_fb_np()
_WR_NP, _WI_NP = _dft_mats_np()


def _fused_body(wav3_ref, wr_ref, wi_ref, fb_ref, wavd_ref,
                mels_ref, oh_ref, code_ref):
    mu = NQUANT - 1
    x = wavd_ref[0]
    xc = jnp.clip(x, -1.0, 1.0)
    amp = jnp.sign(xc) * jnp.log1p(mu * jnp.abs(xc)) / np.log1p(mu)
    code = jnp.floor((amp + 1.0) * 0.5 * mu + 0.5).astype(jnp.int32)
    code_ref[0] = code
    q = lax.broadcasted_iota(jnp.int32, (NQUANT, TB), 0)
    oh_ref[0] = jnp.where(q == code, 1.0, 0.0).astype(jnp.float32)

    @pl.when(pl.program_id(1) == 0)
    def _():
        a = wav3_ref[0]
        a0 = a[0:NFRAMES]
        a1 = a[1:NFRAMES + 1]
        a2 = a[2:NFRAMES + 2]
        f32 = jnp.float32
        re = (jnp.dot(a0, wr_ref[0], preferred_element_type=f32)
              + jnp.dot(a1, wr_ref[1], preferred_element_type=f32)
              + jnp.dot(a2, wr_ref[2], preferred_element_type=f32))
        im = (jnp.dot(a0, wi_ref[0], preferred_element_type=f32)
              + jnp.dot(a1, wi_ref[1], preferred_element_type=f32)
              + jnp.dot(a2, wi_ref[2], preferred_element_type=f32))
        spec = re * re + im * im
        melt = lax.dot_general(fb_ref[...], spec,
                               (((1,), (1,)), ((), ())),
                               preferred_element_type=f32)
        mels_ref[0] = jnp.log(melt + 1e-6)


def kernel(inds_np, wav_np, quant_onehot):
    wav3 = wav_np[:, :102 * HOP].reshape(B, 102, HOP)
    wav_dec = lax.slice(wav_np, (0, L_ENC), (B, T - L_ENC)).reshape(B, 1, TDEC)
    mels, onehot, codes = pl.pallas_call(
        _fused_body,
        grid=(B, NTB),
        in_specs=[
            pl.BlockSpec((1, 102, HOP), lambda b, t: (b, 0, 0)),
            pl.BlockSpec((3, HOP, NBINS), lambda b, t: (0, 0, 0)),
            pl.BlockSpec((3, HOP, NBINS), lambda b, t: (0, 0, 0)),
            pl.BlockSpec((NMELS, NBINS), lambda b, t: (0, 0)),
            pl.BlockSpec((1, 1, TB), lambda b, t: (b, 0, t)),
        ],
        out_specs=[
            pl.BlockSpec((1, NMELS, NFRAMES), lambda b, t: (b, 0, 0)),
            pl.BlockSpec((1, NQUANT, TB), lambda b, t: (b, 0, t)),
            pl.BlockSpec((1, 1, TB), lambda b, t: (b, 0, t)),
        ],
        out_shape=[
            jax.ShapeDtypeStruct((B, NMELS, NFRAMES), jnp.float32),
            jax.ShapeDtypeStruct((B, NQUANT, TDEC), jnp.float32),
            jax.ShapeDtypeStruct((B, 1, TDEC), jnp.int32),
        ],
    )(wav3, jnp.asarray(_WR_NP), jnp.asarray(_WI_NP), jnp.asarray(_FB_NP),
      wav_dec)

    wav_compand_out = lax.slice(codes, (0, 0, L_DEC), (B, 1, TDEC)).reshape(B, TDEC - L_DEC)
    return (inds_np, mels, onehot, wav_compand_out)
